# baseline (speedup 1.0000x reference)
import functools

import jax
import jax.numpy as jnp
from jax import lax
from jax.experimental import pallas as pl
from jax.experimental.pallas import tpu as pltpu
from jax.experimental.pallas import tpu_sc as plsc

_NUM_AGENTS = 4096
_FEAT = 3

_NS = plsc.get_sparse_core_info().num_subcores
_ROWS = _NUM_AGENTS // _NS


def _body(table_hbm, out_hbm, buf):
    sid = lax.axis_index("s")
    r0 = sid * _ROWS
    pltpu.sync_copy(table_hbm.at[pl.ds(r0, _ROWS), :], buf)
    pltpu.sync_copy(buf, out_hbm.at[pl.ds(r0, _ROWS), :])


_sc = functools.partial(
    pl.kernel,
    out_type=jax.ShapeDtypeStruct((_NUM_AGENTS, _FEAT), jnp.float32),
    mesh=plsc.VectorSubcoreMesh(
        core_axis_name="c", subcore_axis_name="s", num_cores=1
    ),
    scratch_types=[pltpu.VMEM((_ROWS, _FEAT), jnp.float32)],
)(_body)


def kernel(pos_phi, num_agents):
    return _sc(pos_phi)
